# baseline (speedup 1.0000x reference)
import functools

import jax
import jax.numpy as jnp
from jax import lax
from jax.experimental import pallas as pl
from jax.experimental.pallas import tpu as pltpu
from jax.experimental.pallas import tpu_sc as plsc


def _proj_body(x3_ref, w_ref, b_ref, o_ref):
    for s in range(8):
        ys = (
            jnp.dot(x3_ref[:, s, :], w_ref[...],
                    preferred_element_type=jnp.float32)
            + b_ref[...]
        )
        o_ref[:, 16 * s:16 * (s + 1)] = ys


def _project_packed(edges, w1t, b):
    E, D = edges.shape
    OUT = w1t.shape[1]
    assert OUT == 16
    G = 1600
    edges3 = edges.reshape(E // 8, 8, D)
    assert (E // 8) % G == 0
    return pl.pallas_call(
        _proj_body,
        grid=(E // 8 // G,),
        in_specs=[
            pl.BlockSpec((G, 8, D), lambda i: (i, 0, 0)),
            pl.BlockSpec((D, OUT), lambda i: (0, 0)),
            pl.BlockSpec((1, OUT), lambda i: (0, 0)),
        ],
        out_specs=pl.BlockSpec((G, 128), lambda i: (i, 0)),
        out_shape=jax.ShapeDtypeStruct((E * OUT // 128, 128), jnp.float32),
    )(edges3, w1t, b.reshape(1, OUT))


def _gather_transpose(table, ids):
    E, OUT = table.shape
    assert OUT == 16
    NB = E // 128
    assert E % 128 == 0
    info = plsc.get_sparse_core_info()
    nw = info.num_cores * info.num_subcores
    S = 8
    CH = S * 128
    NFULL = (NB // nw) // S
    assert NFULL * S * nw <= NB
    mesh = plsc.VectorSubcoreMesh(core_axis_name="c", subcore_axis_name="s")

    @functools.partial(
        pl.kernel,
        mesh=mesh,
        compiler_params=pltpu.CompilerParams(
            use_tc_tiling_on_sc=False, needs_layout_passes=False
        ),
        out_type=jax.ShapeDtypeStruct((4 * NB * 8, 128), jnp.float32),
        scratch_types=[
            pltpu.VMEM((2, CH), jnp.int32),
            pltpu.VMEM((2, CH), jnp.int32),
            pltpu.VMEM((2 * CH, OUT), jnp.float32),
            pltpu.VMEM((2 * CH, OUT), jnp.float32),
            pltpu.VMEM((4 * S * 8, 129), jnp.float32),
            pltpu.VMEM((128,), jnp.int32),
            pltpu.VMEM((128,), jnp.int32),
            pltpu.VMEM((128, OUT), jnp.float32),
            pltpu.VMEM((128, OUT), jnp.float32),
            pltpu.VMEM((4 * 8, 129), jnp.float32),
            pltpu.SemaphoreType.DMA,
            pltpu.SemaphoreType.DMA,
            pltpu.SemaphoreType.DMA,
        ],
    )
    def k(table_hbm, ids_hbm, out_hbm,
          idsf, idsr, rows_f, rows_r, tbuf,
          idsft, idsrt, rows_ft, rows_rt, tbuft,
          semf, semr, semo):
        wid = lax.axis_index("s") * info.num_cores + lax.axis_index("c")
        bs = (wid * NB) // nw
        be = ((wid + 1) * NB) // nw
        iota = lax.iota(jnp.int32, 16)

        def fire(t, d):
            i0 = (bs + S * t) * 128
            pltpu.sync_copy(ids_hbm.at[pl.ds(i0, CH)], idsf.at[d])
            pltpu.sync_copy(ids_hbm.at[pl.ds(E - i0 - CH, CH)], idsr.at[d])
            cf = pltpu.async_copy(table_hbm.at[idsf.at[d]],
                                  rows_f.at[pl.ds(d * CH, CH)], semf)
            cr = pltpu.async_copy(table_hbm.at[idsr.at[d]],
                                  rows_r.at[pl.ds(d * CH, CH)], semr)
            return cf, cr

        h0 = fire(0, 0)
        handles = [h0, None]
        out_handles = None
        rowbase_f = (iota >> 3) * (S * 8) + (iota & 7)
        rowbase_r = rowbase_f + 2 * S * 8
        for t in range(NFULL):
            d = t % 2
            cf, cr = handles[d]
            cf.wait()
            cr.wait()
            if t + 1 < NFULL:
                handles[(t + 1) % 2] = fire(t + 1, (t + 1) % 2)
            if out_handles is not None:
                for h in out_handles:
                    h.wait()
            q0 = d * CH

            @plsc.parallel_loop(0, CH // 8, 1, unroll=4)
            def move(qq):
                for kk in range(8):
                    q = 8 * qq + kk
                    s8 = (q >> 7) * 8
                    ee = q & 127
                    sv = jnp.broadcast_to(s8, (16,))
                    ev = jnp.broadcast_to(ee, (16,))
                    vf = rows_f[q0 + q, :]
                    plsc.store_scatter(tbuf, [rowbase_f + sv, ev], vf)
                    vr = rows_r[q0 + (CH - 1) - q, :]
                    plsc.store_scatter(tbuf, [rowbase_r + sv, ev], vr)

            b0 = bs + S * t
            out_handles = [
                pltpu.async_copy(
                    tbuf.at[pl.ds(a * S * 8, S * 8), pl.ds(0, 128)],
                    out_hbm.at[pl.ds((a * NB + b0) * 8, S * 8)],
                    semo,
                )
                for a in range(4)
            ]
        for h in out_handles:
            h.wait()

        def tailblk(b, _):
            i0 = b * 128
            pltpu.sync_copy(ids_hbm.at[pl.ds(i0, 128)], idsft)
            pltpu.sync_copy(ids_hbm.at[pl.ds(E - i0 - 128, 128)], idsrt)
            cf = pltpu.async_copy(table_hbm.at[idsft], rows_ft, semf)
            cr = pltpu.async_copy(table_hbm.at[idsrt], rows_rt, semr)
            cf.wait()
            cr.wait()

            rowbase_f = (iota >> 3) * 8 + (iota & 7)
            rowbase_r = rowbase_f + 16

            def move(qq, _):
                for kk in range(8):
                    q = 8 * qq + kk
                    ev = jnp.broadcast_to(q, (16,))
                    vf = rows_ft[q, :]
                    plsc.store_scatter(tbuft, [rowbase_f, ev], vf)
                    vr = rows_rt[127 - q, :]
                    plsc.store_scatter(tbuft, [rowbase_r, ev], vr)
                return 0

            lax.fori_loop(0, 16, move, 0)

            for a in range(4):
                pltpu.sync_copy(
                    tbuft.at[pl.ds(a * 8, 8), pl.ds(0, 128)],
                    out_hbm.at[pl.ds((a * NB + b) * 8, 8)],
                )
            return 0

        lax.fori_loop(bs + NFULL * S, be, tailblk, 0)

    return k(table, ids)


def kernel(edges, W, b, edge_ids, combine_method):
    E, D = edges.shape
    OUT = W.shape[0]
    w1t = W[:, :D].T
    proj_packed = _project_packed(edges, w1t, b)
    proj = proj_packed.reshape(E, OUT)
    flat = _gather_transpose(proj, edge_ids)
    x6 = flat.reshape(4, E // 128, 8, 128)
    return x6.transpose(1, 3, 0, 2).reshape(E, 2 * OUT)


# speedup vs baseline: 1.0295x; 1.0295x over previous
import functools

import jax
import jax.numpy as jnp
from jax import lax
from jax.experimental import pallas as pl
from jax.experimental.pallas import tpu as pltpu
from jax.experimental.pallas import tpu_sc as plsc


def _proj_body(x3_ref, w_ref, b_ref, o_ref):
    for s in range(8):
        ys = (
            jnp.dot(x3_ref[:, s, :], w_ref[...],
                    preferred_element_type=jnp.float32)
            + b_ref[...]
        )
        o_ref[:, 16 * s:16 * (s + 1)] = ys


def _project_packed(edges, w1t, b):
    E, D = edges.shape
    OUT = w1t.shape[1]
    assert OUT == 16
    G = 800
    edges3 = edges.reshape(E // 8, 8, D)
    assert (E // 8) % G == 0
    return pl.pallas_call(
        _proj_body,
        grid=(E // 8 // G,),
        in_specs=[
            pl.BlockSpec((G, 8, D), lambda i: (i, 0, 0)),
            pl.BlockSpec((D, OUT), lambda i: (0, 0)),
            pl.BlockSpec((1, OUT), lambda i: (0, 0)),
        ],
        out_specs=pl.BlockSpec((G, 128), lambda i: (i, 0)),
        out_shape=jax.ShapeDtypeStruct((E * OUT // 128, 128), jnp.float32),
    )(edges3, w1t, b.reshape(1, OUT))


def _gather_transpose(table, ids):
    E, OUT = table.shape
    assert OUT == 16
    NB = E // 128
    assert E % 128 == 0
    info = plsc.get_sparse_core_info()
    nw = info.num_cores * info.num_subcores
    S = 8
    CH = S * 128
    NFULL = (NB // nw) // S
    assert NFULL * S * nw <= NB
    mesh = plsc.VectorSubcoreMesh(core_axis_name="c", subcore_axis_name="s")

    @functools.partial(
        pl.kernel,
        mesh=mesh,
        compiler_params=pltpu.CompilerParams(
            use_tc_tiling_on_sc=False, needs_layout_passes=False
        ),
        out_type=jax.ShapeDtypeStruct((4 * NB * 8, 128), jnp.float32),
        scratch_types=[
            pltpu.VMEM((2, CH), jnp.int32),
            pltpu.VMEM((2, CH), jnp.int32),
            pltpu.VMEM((2 * CH, OUT), jnp.float32),
            pltpu.VMEM((2 * CH, OUT), jnp.float32),
            pltpu.VMEM((4 * S * 8, 129), jnp.float32),
            pltpu.VMEM((128,), jnp.int32),
            pltpu.VMEM((128,), jnp.int32),
            pltpu.VMEM((128, OUT), jnp.float32),
            pltpu.VMEM((128, OUT), jnp.float32),
            pltpu.VMEM((4 * 8, 129), jnp.float32),
            pltpu.SemaphoreType.DMA,
            pltpu.SemaphoreType.DMA,
            pltpu.SemaphoreType.DMA,
        ],
    )
    def k(table_hbm, ids_hbm, out_hbm,
          idsf, idsr, rows_f, rows_r, tbuf,
          idsft, idsrt, rows_ft, rows_rt, tbuft,
          semf, semr, semo):
        wid = lax.axis_index("s") * info.num_cores + lax.axis_index("c")
        bs = (wid * NB) // nw
        be = ((wid + 1) * NB) // nw
        iota = lax.iota(jnp.int32, 16)

        def fire(t, d):
            i0 = (bs + S * t) * 128
            pltpu.sync_copy(ids_hbm.at[pl.ds(i0, CH)], idsf.at[d])
            pltpu.sync_copy(ids_hbm.at[pl.ds(E - i0 - CH, CH)], idsr.at[d])
            cf = pltpu.async_copy(table_hbm.at[idsf.at[d]],
                                  rows_f.at[pl.ds(d * CH, CH)], semf)
            cr = pltpu.async_copy(table_hbm.at[idsr.at[d]],
                                  rows_r.at[pl.ds(d * CH, CH)], semr)
            return cf, cr

        h0 = fire(0, 0)
        handles = [h0, None]
        out_handles = None
        rowbase_f = (iota >> 3) * (S * 8) + (iota & 7)
        rowbase_r = rowbase_f + 2 * S * 8
        for t in range(NFULL):
            d = t % 2
            cf, cr = handles[d]
            cf.wait()
            cr.wait()
            if t + 1 < NFULL:
                handles[(t + 1) % 2] = fire(t + 1, (t + 1) % 2)
            if out_handles is not None:
                for h in out_handles:
                    h.wait()
            q0 = d * CH

            @plsc.parallel_loop(0, CH // 8, 1, unroll=2)
            def move(qq):
                for kk in range(8):
                    q = 8 * qq + kk
                    s8 = (q >> 7) * 8
                    ee = q & 127
                    sv = jnp.broadcast_to(s8, (16,))
                    ev = jnp.broadcast_to(ee, (16,))
                    vf = rows_f[q0 + q, :]
                    plsc.store_scatter(tbuf, [rowbase_f + sv, ev], vf)
                    vr = rows_r[q0 + (CH - 1) - q, :]
                    plsc.store_scatter(tbuf, [rowbase_r + sv, ev], vr)

            b0 = bs + S * t
            out_handles = [
                pltpu.async_copy(
                    tbuf.at[pl.ds(a * S * 8, S * 8), pl.ds(0, 128)],
                    out_hbm.at[pl.ds((a * NB + b0) * 8, S * 8)],
                    semo,
                )
                for a in range(4)
            ]
        for h in out_handles:
            h.wait()

        def tailblk(b, _):
            i0 = b * 128
            pltpu.sync_copy(ids_hbm.at[pl.ds(i0, 128)], idsft)
            pltpu.sync_copy(ids_hbm.at[pl.ds(E - i0 - 128, 128)], idsrt)
            cf = pltpu.async_copy(table_hbm.at[idsft], rows_ft, semf)
            cr = pltpu.async_copy(table_hbm.at[idsrt], rows_rt, semr)
            cf.wait()
            cr.wait()

            rowbase_f = (iota >> 3) * 8 + (iota & 7)
            rowbase_r = rowbase_f + 16

            @plsc.parallel_loop(0, 16, 1, unroll=2)
            def move(qq):
                for kk in range(8):
                    q = 8 * qq + kk
                    ev = jnp.broadcast_to(q, (16,))
                    vf = rows_ft[q, :]
                    plsc.store_scatter(tbuft, [rowbase_f, ev], vf)
                    vr = rows_rt[127 - q, :]
                    plsc.store_scatter(tbuft, [rowbase_r, ev], vr)

            for a in range(4):
                pltpu.sync_copy(
                    tbuft.at[pl.ds(a * 8, 8), pl.ds(0, 128)],
                    out_hbm.at[pl.ds((a * NB + b) * 8, 8)],
                )
            return 0

        lax.fori_loop(bs + NFULL * S, be, tailblk, 0)

    return k(table, ids)


def kernel(edges, W, b, edge_ids, combine_method):
    E, D = edges.shape
    OUT = W.shape[0]
    w1t = W[:, :D].T
    proj_packed = _project_packed(edges, w1t, b)
    proj = proj_packed.reshape(E, OUT)
    flat = _gather_transpose(proj, edge_ids)
    x6 = flat.reshape(4, E // 128, 8, 128)
    return x6.transpose(1, 3, 0, 2).reshape(E, 2 * OUT)
